# baseline (speedup 1.0000x reference)
import functools

import jax
import jax.numpy as jnp
from jax import lax
from jax.experimental import pallas as pl
from jax.experimental.pallas import tpu as pltpu, tpu_sc as plsc

_ROWS = 100000
_NW = 32
_RPW = _ROWS // _NW
_CHUNK = 125
_NCHUNK = _RPW // _CHUNK

_SLICES = (
    (0, 0, 64, 0),
    (128, 64, 96, 0),
    (320, 160, 80, 0),
    (64, 0, 64, 1),
    (224, 64, 96, 1),
    (400, 160, 80, 1),
)

_mesh = plsc.VectorSubcoreMesh(core_axis_name="c", subcore_axis_name="s")


@functools.partial(
    pl.kernel,
    mesh=_mesh,
    out_type=(
        jax.ShapeDtypeStruct((_ROWS, 240), jnp.float32),
        jax.ShapeDtypeStruct((_ROWS, 240), jnp.float32),
    ),
    scratch_types=[
        pltpu.VMEM((_CHUNK, 240), jnp.float32),
        pltpu.VMEM((_CHUNK, 240), jnp.float32),
        pltpu.VMEM((_CHUNK, 240), jnp.float32),
        pltpu.VMEM((_CHUNK, 240), jnp.float32),
        pltpu.SemaphoreType.DMA,
        pltpu.SemaphoreType.DMA,
        pltpu.SemaphoreType.DMA,
        pltpu.SemaphoreType.DMA,
    ],
    compiler_params=pltpu.CompilerParams(use_tc_tiling_on_sc=False),
)
def _half_split(x_hbm, out0_hbm, out1_hbm,
                s0b0, s0b1, s1b0, s1b1, si0, si1, so0, so1):
    wid = lax.axis_index("s") * 2 + lax.axis_index("c")
    base = wid * _RPW
    slots = ((s0b0, s0b1, si0, so0), (s1b0, s1b1, si1, so1))
    outs = (out0_hbm, out1_hbm)

    def in_descs(c, slot):
        r0 = base + c * _CHUNK
        b = slots[slot]
        return [
            pltpu.make_async_copy(
                x_hbm.at[pl.ds(r0, _CHUNK), pl.ds(src_col, width)],
                b[oi].at[:, pl.ds(dst_col, width)],
                b[2],
            )
            for src_col, dst_col, width, oi in _SLICES
        ]

    def out_descs(c, slot):
        r0 = base + c * _CHUNK
        b = slots[slot]
        return [
            pltpu.make_async_copy(b[oi], outs[oi].at[pl.ds(r0, _CHUNK), :], b[3])
            for oi in (0, 1)
        ]

    def issue_in(c, slot):
        for d in in_descs(c, slot):
            d.start()

    def wait_in(c, slot):
        for d in in_descs(c, slot):
            d.wait()

    def issue_out(c, slot):
        for d in out_descs(c, slot):
            d.start()

    def wait_out(c, slot):
        for d in out_descs(c, slot):
            d.wait()

    issue_in(0, 0)
    wait_in(0, 0)
    issue_out(0, 0)
    issue_in(1, 1)
    wait_in(1, 1)
    issue_out(1, 1)
    wait_out(0, 0)
    issue_in(2, 0)

    def pair(t, carry):
        c = 2 * t
        wait_in(c, 0)
        issue_out(c, 0)
        wait_out(c - 1, 1)
        issue_in(c + 1, 1)
        wait_in(c + 1, 1)
        issue_out(c + 1, 1)
        wait_out(c, 0)
        issue_in(c + 2, 0)
        return carry

    lax.fori_loop(1, (_NCHUNK - 1) // 2, pair, 0)

    c_last = _NCHUNK - 1
    wait_in(c_last, 0)
    issue_out(c_last, 0)
    wait_out(c_last - 1, 1)
    wait_out(c_last, 0)


def kernel(x):
    return _half_split(x)
